# baseline (speedup 1.0000x reference)
import jax
import jax.numpy as jnp
from jax import lax
from jax.experimental import pallas as pl
from jax.experimental.pallas import tpu as pltpu
from jax.experimental.pallas import tpu_sc as plsc

N = 50000
E = 800000
H = 64
HQ = 16
G = 64
CH = 128
EPAD = 819200
ROWS = EPAD // CH
CPT = ROWS // 16
EPT = EPAD // 16
MC = 512
KB = 10240
NC = KB // MC
NB = EPT // KB
NBUF = 4
PG = 2
PS = NBUF - PG
ACC_ROWS = 50176
ZPT = ACC_ROWS // 16
BLK = 1000
GRID = N // BLK

_MESH = plsc.VectorSubcoreMesh(core_axis_name="c", subcore_axis_name="s")
_SC_PARAMS = pltpu.CompilerParams(use_tc_tiling_on_sc=False)


def _memset_rows(buf, rows, value):
  v = jnp.full((16,), value, jnp.float32)

  def row(i, _):
    buf[i, pl.ds(0, 16)] = v
    return 0

  lax.fori_loop(0, rows, row, 0)


def _zero_acc_slice(acc, zbuf, base):
  for k in range(ZPT // CH):
    pltpu.sync_copy(zbuf, acc.at[pl.ds(base + k * CH, CH)])
  rem = ZPT - (ZPT // CH) * CH
  if rem:
    pltpu.sync_copy(zbuf.at[pl.ds(0, rem)],
                    acc.at[pl.ds(base + (ZPT // CH) * CH, rem)])


def _sc_layer_body(h0, h1, h2, h3, src_i, dst_i, o0, o1, o2, o3,
                   srcb, dstb, gbuf, acc, gsem, ssem):
  c = lax.axis_index("c")
  s = lax.axis_index("s")

  def run(h_ref, out_ref):
    _memset_rows(gbuf.at[0], CH, 0.0)
    _zero_acc_slice(acc, gbuf.at[0, pl.ds(0, CH)], s * ZPT)
    plsc.subcore_barrier()

    def gather(k):
      pltpu.async_copy(h_ref.at[srcb.at[pl.ds(k * MC, MC)]],
                       gbuf.at[lax.rem(k, NBUF)], gsem)

    def scatter(k):
      pltpu.async_copy(gbuf.at[lax.rem(k, NBUF)],
                       acc.at[dstb.at[pl.ds(k * MC, MC)]], ssem, add=True)

    def wait_g():
      pltpu.make_async_copy(h_ref.at[srcb.at[pl.ds(0, MC)]], gbuf.at[0],
                            gsem).wait()

    def wait_s():
      pltpu.make_async_copy(gbuf.at[0], acc.at[dstb.at[pl.ds(0, MC)]],
                            ssem).wait()

    for blk in range(NB):
      base = s * EPT + blk * KB
      pltpu.sync_copy(src_i.at[pl.ds(base, KB)], srcb)
      pltpu.sync_copy(dst_i.at[pl.ds(base, KB)], dstb)

      for k in range(PG):
        gather(k)

      def body1(k, _):
        wait_g()
        scatter(k)
        gather(k + PG)
        return 0

      def body2(k, _):
        wait_g()
        scatter(k)
        wait_s()
        gather(k + PG)
        return 0

      def body3(k, _):
        wait_g()
        scatter(k)
        wait_s()
        return 0

      lax.fori_loop(0, PS, body1, 0)
      lax.fori_loop(PS, NC - PG, body2, 0, unroll=2)
      lax.fori_loop(NC - PG, NC, body3, 0)
      for _ in range(PS):
        wait_s()

    plsc.subcore_barrier()
    pltpu.sync_copy(acc.at[pl.ds(s * ZPT, ZPT)], out_ref.at[pl.ds(s * ZPT, ZPT)])
    plsc.subcore_barrier()

  @pl.when(c == 0)
  def _():
    run(h0, o0)
    run(h1, o1)

  @pl.when(c == 1)
  def _():
    run(h2, o2)
    run(h3, o3)


_sc_layer = pl.kernel(
    _sc_layer_body,
    out_type=[jax.ShapeDtypeStruct((ACC_ROWS, HQ), jnp.float32)] * 4,
    mesh=_MESH,
    scratch_types=[
        pltpu.VMEM((KB,), jnp.int32),
        pltpu.VMEM((KB,), jnp.int32),
        pltpu.VMEM((NBUF, MC, HQ), jnp.float32),
        pltpu.VMEM_SHARED((ACC_ROWS, HQ), jnp.float32),
        pltpu.SemaphoreType.DMA,
        pltpu.SemaphoreType.DMA,
    ],
    compiler_params=_SC_PARAMS,
)


def _split4(ref_list, val):
  for q, r in enumerate(ref_list):
    r[...] = val[:, q * HQ:(q + 1) * HQ]


def _tc_mid_body(s0, s1, s2, s3, c0, c1, c2, c3, dpc, x_ref, w1_ref,
                 wn_ref, b_ref, ff_ref, lf_ref, o0, o1, o2, o3, dpo):
  ff = ff_ref[...] > 0.0
  lf = lf_ref[...] > 0.0
  deg_cols = jnp.where(ff, s0[...], dpc[...])
  dpo[...] = deg_cols
  dinv = lax.rsqrt(1.0 + deg_cols[:, :1])
  hp = jnp.dot(x_ref[...], w1_ref[...],
               preferred_element_type=jnp.float32) * dinv
  z = jnp.concatenate(
      [s[...] + c[...] for s, c in zip((s0, s1, s2, s3), (c0, c1, c2, c3))],
      axis=1)
  z = z * dinv + b_ref[...]
  a = jnp.maximum(z, 0.0)
  y = jnp.dot(a, wn_ref[...], preferred_element_type=jnp.float32) * dinv
  out = jnp.where(ff, hp, jnp.where(lf, z, y))
  _split4([o0, o1, o2, o3], out)


def _tc_pool_body(h0, h1, h2, h3, batch_ref, h_out, pooled, sums, cnt):
  i = pl.program_id(0)
  z = jnp.concatenate([h0[...], h1[...], h2[...], h3[...]], axis=1)
  h_out[...] = z
  bt = batch_ref[0]
  iota = lax.broadcasted_iota(jnp.int32, (G, BLK), 0)
  msk = (iota == bt).astype(jnp.float32)

  @pl.when(i == 0)
  def _():
    sums[...] = jnp.zeros_like(sums)
    cnt[...] = jnp.zeros_like(cnt)

  sums[...] += jnp.dot(msk, z, preferred_element_type=jnp.float32)
  cnt[...] += jnp.dot(msk, jnp.ones((BLK, 1), jnp.float32),
                      preferred_element_type=jnp.float32)
  pooled[...] = sums[...] / jnp.maximum(cnt[...], 1.0)


def _row_spec(w):
  return pl.BlockSpec((BLK, w), lambda i: (i, 0))


def _full_spec(shape):
  return pl.BlockSpec(shape, lambda i: tuple(0 for _ in shape))


_Q = [jax.ShapeDtypeStruct((N, HQ), jnp.float32)] * 4

_tc_mid = pl.pallas_call(
    _tc_mid_body,
    grid=(GRID,),
    in_specs=[_row_spec(HQ)] * 8 + [_row_spec(16), _row_spec(8),
              _full_spec((8, H)), _full_spec((H, H)), _full_spec((1, H)),
              _full_spec((1, 1)), _full_spec((1, 1))],
    out_specs=[_row_spec(HQ)] * 5,
    out_shape=_Q + [jax.ShapeDtypeStruct((N, 16), jnp.float32)],
)

_tc_pool = pl.pallas_call(
    _tc_pool_body,
    grid=(GRID,),
    in_specs=[_row_spec(HQ)] * 4 +
             [pl.BlockSpec((1, 1, BLK), lambda i: (i, 0, 0))],
    out_specs=[_row_spec(H), _full_spec((G, H))],
    out_shape=[jax.ShapeDtypeStruct((N, H), jnp.float32),
               jax.ShapeDtypeStruct((G, H), jnp.float32)],
    scratch_shapes=[pltpu.VMEM((G, H), jnp.float32),
                    pltpu.VMEM((G, 1), jnp.float32)],
)


@jax.jit
def kernel(x, edge_index, batch, W1, b1, W2, b2, W3, b3):
  src = edge_index[0]
  dst = edge_index[1]
  pad = EPAD - E
  src_i = jnp.concatenate([src, jnp.zeros((pad,), jnp.int32)])
  dst_i = jnp.concatenate([dst, jnp.full((pad,), N, jnp.int32)])
  xp = jnp.pad(x, ((0, 0), (0, 1)))
  W1p = jnp.pad(W1, ((0, 1), (0, 0)))

  ones_q = jnp.ones((N, HQ), jnp.float32)
  carry = (ones_q, ones_q, ones_q, ones_q, jnp.zeros((N, 16), jnp.float32))

  zH = jnp.zeros((H, H), jnp.float32)
  z1 = jnp.zeros((1, H), jnp.float32)
  wn_stack = jnp.stack([zH, W2, W3, zH])
  b_stack = jnp.stack([z1, b1.reshape(1, H), b2.reshape(1, H),
                       b3.reshape(1, H)])
  ff = jnp.array([1.0, 0.0, 0.0, 0.0], jnp.float32).reshape(4, 1, 1)
  lf = jnp.array([0.0, 0.0, 0.0, 1.0], jnp.float32).reshape(4, 1, 1)

  def body(carry, xs):
    wn, b, f0, f1 = xs
    sq = _sc_layer(*carry[:4], src_i, dst_i)
    nxt = _tc_mid(*sq, *carry, xp, W1p, wn, b, f0, f1)
    return tuple(nxt), None

  carry, _ = lax.scan(body, carry, (wn_stack, b_stack, ff, lf))
  h, pooled = _tc_pool(*carry[:4], batch.reshape(GRID, 1, BLK))
  return (h, pooled)


# speedup vs baseline: 1.9069x; 1.9069x over previous
import jax
import jax.numpy as jnp
from jax import lax
from jax.experimental import pallas as pl
from jax.experimental.pallas import tpu as pltpu
from jax.experimental.pallas import tpu_sc as plsc

N = 50000
E = 800000
H = 64
HH = 32
HQ = 16
G = 64
CH = 128
EPAD = 819200
ROWS = EPAD // CH
CPT = ROWS // 16
EPT = EPAD // 16
MC = 512
KB = 10240
NC = KB // MC
NB = EPT // KB
NBUF = 4
PG = 2
PS = NBUF - PG
ACC_ROWS = 50176
ZPT = ACC_ROWS // 16
BLK = 1000
GRID = N // BLK

_MESH = plsc.VectorSubcoreMesh(core_axis_name="c", subcore_axis_name="s")
_SC_PARAMS = pltpu.CompilerParams(use_tc_tiling_on_sc=False)


def _memset_rows(buf, rows, value):
  v = jnp.full((32,), value, jnp.bfloat16)

  def row(i, _):
    buf[i, pl.ds(0, 32)] = v
    return 0

  lax.fori_loop(0, rows, row, 0)


def _zero_acc_slice(acc, zbuf, base):
  for k in range(ZPT // CH):
    pltpu.sync_copy(zbuf, acc.at[pl.ds(base + k * CH, CH)])
  rem = ZPT - (ZPT // CH) * CH
  if rem:
    pltpu.sync_copy(zbuf.at[pl.ds(0, rem)],
                    acc.at[pl.ds(base + (ZPT // CH) * CH, rem)])


def _sc_layer_body(h0, h1, src_i, dst_i, o0, o1,
                   srcb, dstb, gbuf, acc, gsem, ssem):
  c = lax.axis_index("c")
  s = lax.axis_index("s")

  def run(h_ref, out_ref):
    _memset_rows(gbuf.at[0], CH, 0.0)
    _zero_acc_slice(acc, gbuf.at[0, pl.ds(0, CH)], s * ZPT)
    plsc.subcore_barrier()

    def gather(k):
      pltpu.async_copy(h_ref.at[srcb.at[pl.ds(k * MC, MC)]],
                       gbuf.at[lax.rem(k, NBUF)], gsem)

    def scatter(k):
      pltpu.async_copy(gbuf.at[lax.rem(k, NBUF)],
                       acc.at[dstb.at[pl.ds(k * MC, MC)]], ssem, add=True)

    def wait_g():
      pltpu.make_async_copy(h_ref.at[srcb.at[pl.ds(0, MC)]], gbuf.at[0],
                            gsem).wait()

    def wait_s():
      pltpu.make_async_copy(gbuf.at[0], acc.at[dstb.at[pl.ds(0, MC)]],
                            ssem).wait()

    for blk in range(NB):
      base = s * EPT + blk * KB
      pltpu.sync_copy(src_i.at[pl.ds(base, KB)], srcb)
      pltpu.sync_copy(dst_i.at[pl.ds(base, KB)], dstb)

      for k in range(PG):
        gather(k)

      def body1(k, _):
        wait_g()
        scatter(k)
        gather(k + PG)
        return 0

      def body2(k, _):
        wait_g()
        scatter(k)
        wait_s()
        gather(k + PG)
        return 0

      def body3(k, _):
        wait_g()
        scatter(k)
        wait_s()
        return 0

      lax.fori_loop(0, PS, body1, 0)
      lax.fori_loop(PS, NC - PG, body2, 0, unroll=2)
      lax.fori_loop(NC - PG, NC, body3, 0)
      for _ in range(PS):
        wait_s()

    plsc.subcore_barrier()
    pltpu.sync_copy(acc.at[pl.ds(s * ZPT, ZPT)], out_ref.at[pl.ds(s * ZPT, ZPT)])
    plsc.subcore_barrier()

  @pl.when(c == 0)
  def _():
    run(h0, o0)

  @pl.when(c == 1)
  def _():
    run(h1, o1)


_sc_layer = pl.kernel(
    _sc_layer_body,
    out_type=[jax.ShapeDtypeStruct((ACC_ROWS, HH), jnp.bfloat16)] * 2,
    mesh=_MESH,
    scratch_types=[
        pltpu.VMEM((KB,), jnp.int32),
        pltpu.VMEM((KB,), jnp.int32),
        pltpu.VMEM((NBUF, MC, HH), jnp.bfloat16),
        pltpu.VMEM_SHARED((ACC_ROWS, HH), jnp.bfloat16),
        pltpu.SemaphoreType.DMA,
        pltpu.SemaphoreType.DMA,
    ],
    compiler_params=_SC_PARAMS,
)


def _tc_mid_body(s0, s1, c0, c1, dpc, x_ref, w1_ref,
                 wn_ref, b_ref, ff_ref, lf_ref, o0, o1, dpo):
  ff = ff_ref[...] > 0.0
  lf = lf_ref[...] > 0.0
  s0f = s0[...].astype(jnp.float32)
  s1f = s1[...].astype(jnp.float32)
  deg_cols = jnp.where(ff, s0f[:, :HQ], dpc[...])
  dpo[...] = deg_cols
  dinv = lax.rsqrt(1.0 + deg_cols[:, :1])
  hp = jnp.dot(x_ref[...], w1_ref[...],
               preferred_element_type=jnp.float32) * dinv
  z = jnp.concatenate(
      [s0f + c0[...].astype(jnp.float32), s1f + c1[...].astype(jnp.float32)],
      axis=1)
  z = z * dinv + b_ref[...]
  a = jnp.maximum(z, 0.0)
  y = jnp.dot(a, wn_ref[...], preferred_element_type=jnp.float32) * dinv
  out = jnp.where(ff, hp, jnp.where(lf, z, y))
  o0[...] = out[:, :HH].astype(jnp.bfloat16)
  o1[...] = out[:, HH:].astype(jnp.bfloat16)


def _tc_pool_body(h0, h1, batch_ref, h_out, pooled, sums, cnt):
  i = pl.program_id(0)
  z = jnp.concatenate([h0[...].astype(jnp.float32),
                       h1[...].astype(jnp.float32)], axis=1)
  h_out[...] = z
  bt = batch_ref[0]
  iota = lax.broadcasted_iota(jnp.int32, (G, BLK), 0)
  msk = (iota == bt).astype(jnp.float32)

  @pl.when(i == 0)
  def _():
    sums[...] = jnp.zeros_like(sums)
    cnt[...] = jnp.zeros_like(cnt)

  sums[...] += jnp.dot(msk, z, preferred_element_type=jnp.float32)
  cnt[...] += jnp.dot(msk, jnp.ones((BLK, 1), jnp.float32),
                      preferred_element_type=jnp.float32)
  pooled[...] = sums[...] / jnp.maximum(cnt[...], 1.0)


def _row_spec(w):
  return pl.BlockSpec((BLK, w), lambda i: (i, 0))


def _full_spec(shape):
  return pl.BlockSpec(shape, lambda i: tuple(0 for _ in shape))


_HB = [jax.ShapeDtypeStruct((N, HH), jnp.bfloat16)] * 2

_tc_mid = pl.pallas_call(
    _tc_mid_body,
    grid=(GRID,),
    in_specs=[_row_spec(HH)] * 4 + [_row_spec(16), _row_spec(8),
              _full_spec((8, H)), _full_spec((H, H)), _full_spec((1, H)),
              _full_spec((1, 1)), _full_spec((1, 1))],
    out_specs=[_row_spec(HH)] * 2 + [_row_spec(16)],
    out_shape=_HB + [jax.ShapeDtypeStruct((N, 16), jnp.float32)],
)

_tc_pool = pl.pallas_call(
    _tc_pool_body,
    grid=(GRID,),
    in_specs=[_row_spec(HH)] * 2 +
             [pl.BlockSpec((1, 1, BLK), lambda i: (i, 0, 0))],
    out_specs=[_row_spec(H), _full_spec((G, H))],
    out_shape=[jax.ShapeDtypeStruct((N, H), jnp.float32),
               jax.ShapeDtypeStruct((G, H), jnp.float32)],
    scratch_shapes=[pltpu.VMEM((G, H), jnp.float32),
                    pltpu.VMEM((G, 1), jnp.float32)],
)


@jax.jit
def kernel(x, edge_index, batch, W1, b1, W2, b2, W3, b3):
  src = edge_index[0]
  dst = edge_index[1]
  pad = EPAD - E
  src_i = jnp.concatenate([src, jnp.zeros((pad,), jnp.int32)])
  dst_i = jnp.concatenate([dst, jnp.full((pad,), N, jnp.int32)])
  xp = jnp.pad(x, ((0, 0), (0, 1)))
  W1p = jnp.pad(W1, ((0, 1), (0, 0)))

  ones_h = jnp.ones((N, HH), jnp.bfloat16)
  carry = (ones_h, ones_h, jnp.zeros((N, 16), jnp.float32))

  zH = jnp.zeros((H, H), jnp.float32)
  z1 = jnp.zeros((1, H), jnp.float32)
  wn_stack = jnp.stack([zH, W2, W3, zH])
  b_stack = jnp.stack([z1, b1.reshape(1, H), b2.reshape(1, H),
                       b3.reshape(1, H)])
  ff = jnp.array([1.0, 0.0, 0.0, 0.0], jnp.float32).reshape(4, 1, 1)
  lf = jnp.array([0.0, 0.0, 0.0, 1.0], jnp.float32).reshape(4, 1, 1)

  def body(carry, xs):
    wn, b, f0, f1 = xs
    sq = _sc_layer(*carry[:2], src_i, dst_i)
    nxt = _tc_mid(*sq, *carry, xp, W1p, wn, b, f0, f1)
    return tuple(nxt), None

  carry, _ = lax.scan(body, carry, (wn_stack, b_stack, ff, lf))
  h, pooled = _tc_pool(*carry[:2], batch.reshape(GRID, 1, BLK))
  return (h, pooled)


# speedup vs baseline: 1.9740x; 1.0352x over previous
import jax
import jax.numpy as jnp
from jax import lax
from jax.experimental import pallas as pl
from jax.experimental.pallas import tpu as pltpu
from jax.experimental.pallas import tpu_sc as plsc

N = 50000
E = 800000
H = 64
HH = 32
HQ = 16
G = 64
CH = 128
EPAD = 819200
ROWS = EPAD // CH
CPT = ROWS // 16
EPT = EPAD // 16
MC = 512
KB = 10240
NC = KB // MC
NB = EPT // KB
NBUF = 6
PG = 3
PS = NBUF - PG
ACC_ROWS = 50176
ZPT = ACC_ROWS // 16
BLK = 1000
GRID = N // BLK

_MESH = plsc.VectorSubcoreMesh(core_axis_name="c", subcore_axis_name="s")
_SC_PARAMS = pltpu.CompilerParams(use_tc_tiling_on_sc=False)


def _memset_rows(buf, rows, value):
  v = jnp.full((32,), value, jnp.bfloat16)

  def row(i, _):
    buf[i, pl.ds(0, 32)] = v
    return 0

  lax.fori_loop(0, rows, row, 0)


def _zero_acc_slice(acc, zbuf, base):
  for k in range(ZPT // CH):
    pltpu.sync_copy(zbuf, acc.at[pl.ds(base + k * CH, CH)])
  rem = ZPT - (ZPT // CH) * CH
  if rem:
    pltpu.sync_copy(zbuf.at[pl.ds(0, rem)],
                    acc.at[pl.ds(base + (ZPT // CH) * CH, rem)])


def _sc_layer_body(h0, h1, src_i, dst_i, o0, o1,
                   srcb, dstb, gbuf, acc, gsem, ssem):
  c = lax.axis_index("c")
  s = lax.axis_index("s")

  def run(h_ref, out_ref):
    _memset_rows(gbuf.at[0], CH, 0.0)
    _zero_acc_slice(acc, gbuf.at[0, pl.ds(0, CH)], s * ZPT)
    plsc.subcore_barrier()

    def gather(k):
      pltpu.async_copy(h_ref.at[srcb.at[pl.ds(k * MC, MC)]],
                       gbuf.at[lax.rem(k, NBUF)], gsem)

    def scatter(k):
      pltpu.async_copy(gbuf.at[lax.rem(k, NBUF)],
                       acc.at[dstb.at[pl.ds(k * MC, MC)]], ssem, add=True)

    def wait_g():
      pltpu.make_async_copy(h_ref.at[srcb.at[pl.ds(0, MC)]], gbuf.at[0],
                            gsem).wait()

    def wait_s():
      pltpu.make_async_copy(gbuf.at[0], acc.at[dstb.at[pl.ds(0, MC)]],
                            ssem).wait()

    for blk in range(NB):
      base = s * EPT + blk * KB
      pltpu.sync_copy(src_i.at[pl.ds(base, KB)], srcb)
      pltpu.sync_copy(dst_i.at[pl.ds(base, KB)], dstb)

      for k in range(PG):
        gather(k)

      def body1(k, _):
        wait_g()
        scatter(k)
        gather(k + PG)
        return 0

      def body2(k, _):
        wait_g()
        scatter(k)
        wait_s()
        gather(k + PG)
        return 0

      def body3(k, _):
        wait_g()
        scatter(k)
        wait_s()
        return 0

      lax.fori_loop(0, PS, body1, 0)
      lax.fori_loop(PS, NC - PG, body2, 0, unroll=2)
      lax.fori_loop(NC - PG, NC, body3, 0)
      for _ in range(PS):
        wait_s()

    plsc.subcore_barrier()
    pltpu.sync_copy(acc.at[pl.ds(s * ZPT, ZPT)], out_ref.at[pl.ds(s * ZPT, ZPT)])
    plsc.subcore_barrier()

  @pl.when(c == 0)
  def _():
    run(h0, o0)

  @pl.when(c == 1)
  def _():
    run(h1, o1)


_sc_layer = pl.kernel(
    _sc_layer_body,
    out_type=[jax.ShapeDtypeStruct((ACC_ROWS, HH), jnp.bfloat16)] * 2,
    mesh=_MESH,
    scratch_types=[
        pltpu.VMEM((KB,), jnp.int32),
        pltpu.VMEM((KB,), jnp.int32),
        pltpu.VMEM((NBUF, MC, HH), jnp.bfloat16),
        pltpu.VMEM_SHARED((ACC_ROWS, HH), jnp.bfloat16),
        pltpu.SemaphoreType.DMA,
        pltpu.SemaphoreType.DMA,
    ],
    compiler_params=_SC_PARAMS,
)


def _tc_mid_body(s0, s1, c0, c1, dpc, x_ref, w1_ref,
                 wn_ref, b_ref, ff_ref, lf_ref, o0, o1, dpo):
  ff = ff_ref[...] > 0.0
  lf = lf_ref[...] > 0.0
  s0f = s0[...].astype(jnp.float32)
  s1f = s1[...].astype(jnp.float32)
  deg_cols = jnp.where(ff, s0f[:, :HQ], dpc[...])
  dpo[...] = deg_cols
  dinv = lax.rsqrt(1.0 + deg_cols[:, :1])
  hp = jnp.dot(x_ref[...], w1_ref[...],
               preferred_element_type=jnp.float32) * dinv
  z = jnp.concatenate(
      [s0f + c0[...].astype(jnp.float32), s1f + c1[...].astype(jnp.float32)],
      axis=1)
  z = z * dinv + b_ref[...]
  a = jnp.maximum(z, 0.0)
  y = jnp.dot(a, wn_ref[...], preferred_element_type=jnp.float32) * dinv
  out = jnp.where(ff, hp, jnp.where(lf, z, y))
  o0[...] = out[:, :HH].astype(jnp.bfloat16)
  o1[...] = out[:, HH:].astype(jnp.bfloat16)


def _tc_pool_body(h0, h1, batch_ref, h_out, pooled, sums, cnt):
  i = pl.program_id(0)
  z = jnp.concatenate([h0[...].astype(jnp.float32),
                       h1[...].astype(jnp.float32)], axis=1)
  h_out[...] = z
  bt = batch_ref[0]
  iota = lax.broadcasted_iota(jnp.int32, (G, BLK), 0)
  msk = (iota == bt).astype(jnp.float32)

  @pl.when(i == 0)
  def _():
    sums[...] = jnp.zeros_like(sums)
    cnt[...] = jnp.zeros_like(cnt)

  sums[...] += jnp.dot(msk, z, preferred_element_type=jnp.float32)
  cnt[...] += jnp.dot(msk, jnp.ones((BLK, 1), jnp.float32),
                      preferred_element_type=jnp.float32)
  pooled[...] = sums[...] / jnp.maximum(cnt[...], 1.0)


def _row_spec(w):
  return pl.BlockSpec((BLK, w), lambda i: (i, 0))


def _full_spec(shape):
  return pl.BlockSpec(shape, lambda i: tuple(0 for _ in shape))


_HB = [jax.ShapeDtypeStruct((N, HH), jnp.bfloat16)] * 2

_tc_mid = pl.pallas_call(
    _tc_mid_body,
    grid=(GRID,),
    in_specs=[_row_spec(HH)] * 4 + [_row_spec(16), _row_spec(8),
              _full_spec((8, H)), _full_spec((H, H)), _full_spec((1, H)),
              _full_spec((1, 1)), _full_spec((1, 1))],
    out_specs=[_row_spec(HH)] * 2 + [_row_spec(16)],
    out_shape=_HB + [jax.ShapeDtypeStruct((N, 16), jnp.float32)],
)

_tc_pool = pl.pallas_call(
    _tc_pool_body,
    grid=(GRID,),
    in_specs=[_row_spec(HH)] * 2 +
             [pl.BlockSpec((1, 1, BLK), lambda i: (i, 0, 0))],
    out_specs=[_row_spec(H), _full_spec((G, H))],
    out_shape=[jax.ShapeDtypeStruct((N, H), jnp.float32),
               jax.ShapeDtypeStruct((G, H), jnp.float32)],
    scratch_shapes=[pltpu.VMEM((G, H), jnp.float32),
                    pltpu.VMEM((G, 1), jnp.float32)],
)


@jax.jit
def kernel(x, edge_index, batch, W1, b1, W2, b2, W3, b3):
  src = edge_index[0]
  dst = edge_index[1]
  pad = EPAD - E
  src_i = jnp.concatenate([src, jnp.zeros((pad,), jnp.int32)])
  dst_i = jnp.concatenate([dst, jnp.full((pad,), N, jnp.int32)])
  xp = jnp.pad(x, ((0, 0), (0, 1)))
  W1p = jnp.pad(W1, ((0, 1), (0, 0)))

  ones_h = jnp.ones((N, HH), jnp.bfloat16)
  carry = (ones_h, ones_h, jnp.zeros((N, 16), jnp.float32))

  zH = jnp.zeros((H, H), jnp.float32)
  z1 = jnp.zeros((1, H), jnp.float32)
  wn_stack = jnp.stack([zH, W2, W3, zH])
  b_stack = jnp.stack([z1, b1.reshape(1, H), b2.reshape(1, H),
                       b3.reshape(1, H)])
  ff = jnp.array([1.0, 0.0, 0.0, 0.0], jnp.float32).reshape(4, 1, 1)
  lf = jnp.array([0.0, 0.0, 0.0, 1.0], jnp.float32).reshape(4, 1, 1)

  def body(carry, xs):
    wn, b, f0, f1 = xs
    sq = _sc_layer(*carry[:2], src_i, dst_i)
    nxt = _tc_mid(*sq, *carry, xp, W1p, wn, b, f0, f1)
    return tuple(nxt), None

  carry, _ = lax.scan(body, carry, (wn_stack, b_stack, ff, lf))
  h, pooled = _tc_pool(*carry[:2], batch.reshape(GRID, 1, BLK))
  return (h, pooled)


# speedup vs baseline: 2.0760x; 1.0517x over previous
import jax
import jax.numpy as jnp
from jax import lax
from jax.experimental import pallas as pl
from jax.experimental.pallas import tpu as pltpu
from jax.experimental.pallas import tpu_sc as plsc

N = 50000
E = 800000
H = 64
HH = 32
HQ = 16
G = 64
CH = 128
EPAD = 819200
ROWS = EPAD // CH
CPT = ROWS // 16
EPT = EPAD // 16
MC = 256
KB = 10240
NC = KB // MC
NB = EPT // KB
NBUF = 12
PG = 6
PS = NBUF - PG
ACC_ROWS = 50176
ZPT = ACC_ROWS // 16
BLK = 2000
GRID = N // BLK

_MESH = plsc.VectorSubcoreMesh(core_axis_name="c", subcore_axis_name="s")
_SC_PARAMS = pltpu.CompilerParams(use_tc_tiling_on_sc=False)


def _memset_rows(buf, rows, value):
  v = jnp.full((32,), value, jnp.bfloat16)

  def row(i, _):
    buf[i, pl.ds(0, 32)] = v
    return 0

  lax.fori_loop(0, rows, row, 0)


def _zero_acc_slice(acc, zbuf, base):
  for k in range(ZPT // CH):
    pltpu.sync_copy(zbuf, acc.at[pl.ds(base + k * CH, CH)])
  rem = ZPT - (ZPT // CH) * CH
  if rem:
    pltpu.sync_copy(zbuf.at[pl.ds(0, rem)],
                    acc.at[pl.ds(base + (ZPT // CH) * CH, rem)])


def _sc_layer_body(h0, h1, src_i, dst_i, o0, o1,
                   srcb, dstb, gbuf, acc, gsem, ssem):
  c = lax.axis_index("c")
  s = lax.axis_index("s")

  def run(h_ref, out_ref):
    _memset_rows(gbuf.at[0], CH, 0.0)
    _zero_acc_slice(acc, gbuf.at[0, pl.ds(0, CH)], s * ZPT)
    plsc.subcore_barrier()

    def gather(k):
      pltpu.async_copy(h_ref.at[srcb.at[pl.ds(k * MC, MC)]],
                       gbuf.at[lax.rem(k, NBUF)], gsem)

    def scatter(k):
      pltpu.async_copy(gbuf.at[lax.rem(k, NBUF)],
                       acc.at[dstb.at[pl.ds(k * MC, MC)]], ssem, add=True)

    def wait_g():
      pltpu.make_async_copy(h_ref.at[srcb.at[pl.ds(0, MC)]], gbuf.at[0],
                            gsem).wait()

    def wait_s():
      pltpu.make_async_copy(gbuf.at[0], acc.at[dstb.at[pl.ds(0, MC)]],
                            ssem).wait()

    for blk in range(NB):
      base = s * EPT + blk * KB
      pltpu.sync_copy(src_i.at[pl.ds(base, KB)], srcb)
      pltpu.sync_copy(dst_i.at[pl.ds(base, KB)], dstb)

      for k in range(PG):
        gather(k)

      def body1(k, _):
        wait_g()
        scatter(k)
        gather(k + PG)
        return 0

      def body2(k, _):
        wait_g()
        scatter(k)
        wait_s()
        gather(k + PG)
        return 0

      def body3(k, _):
        wait_g()
        scatter(k)
        wait_s()
        return 0

      lax.fori_loop(0, PS, body1, 0)
      lax.fori_loop(PS, NC - PG, body2, 0, unroll=2)
      lax.fori_loop(NC - PG, NC, body3, 0)
      for _ in range(PS):
        wait_s()

    plsc.subcore_barrier()
    pltpu.sync_copy(acc.at[pl.ds(s * ZPT, ZPT)], out_ref.at[pl.ds(s * ZPT, ZPT)])
    plsc.subcore_barrier()

  @pl.when(c == 0)
  def _():
    run(h0, o0)

  @pl.when(c == 1)
  def _():
    run(h1, o1)


_sc_layer = pl.kernel(
    _sc_layer_body,
    out_type=[jax.ShapeDtypeStruct((ACC_ROWS, HH), jnp.bfloat16)] * 2,
    mesh=_MESH,
    scratch_types=[
        pltpu.VMEM((KB,), jnp.int32),
        pltpu.VMEM((KB,), jnp.int32),
        pltpu.VMEM((NBUF, MC, HH), jnp.bfloat16),
        pltpu.VMEM_SHARED((ACC_ROWS, HH), jnp.bfloat16),
        pltpu.SemaphoreType.DMA,
        pltpu.SemaphoreType.DMA,
    ],
    compiler_params=_SC_PARAMS,
)


def _tc_mid_body(s0, s1, c0, c1, dpc, x_ref, w1_ref,
                 wn_ref, b_ref, ff_ref, lf_ref, o0, o1, dpo):
  ff = ff_ref[...] > 0.0
  lf = lf_ref[...] > 0.0
  s0f = s0[...].astype(jnp.float32)
  s1f = s1[...].astype(jnp.float32)
  deg_cols = jnp.where(ff, s0f[:, :8], dpc[...])
  dpo[...] = deg_cols
  dinv = lax.rsqrt(1.0 + deg_cols[:, :1])
  hp = jnp.dot(x_ref[...], w1_ref[...],
               preferred_element_type=jnp.float32) * dinv
  z = jnp.concatenate(
      [s0f + c0[...].astype(jnp.float32), s1f + c1[...].astype(jnp.float32)],
      axis=1)
  z = z * dinv + b_ref[...]
  a = jnp.maximum(z, 0.0)
  y = jnp.dot(a, wn_ref[...], preferred_element_type=jnp.float32) * dinv
  out = jnp.where(ff, hp, jnp.where(lf, z, y))
  o0[...] = out[:, :HH].astype(jnp.bfloat16)
  o1[...] = out[:, HH:].astype(jnp.bfloat16)


def _tc_pool_body(h0, h1, batch_ref, h_out, pooled, sums, cnt):
  i = pl.program_id(0)
  z = jnp.concatenate([h0[...].astype(jnp.float32),
                       h1[...].astype(jnp.float32)], axis=1)
  h_out[...] = z
  bt = batch_ref[0]
  iota = lax.broadcasted_iota(jnp.int32, (G, BLK), 0)
  msk = (iota == bt).astype(jnp.float32)

  @pl.when(i == 0)
  def _():
    sums[...] = jnp.zeros_like(sums)
    cnt[...] = jnp.zeros_like(cnt)

  sums[...] += jnp.dot(msk, z, preferred_element_type=jnp.float32)
  cnt[...] += jnp.dot(msk, jnp.ones((BLK, 1), jnp.float32),
                      preferred_element_type=jnp.float32)
  pooled[...] = sums[...] / jnp.maximum(cnt[...], 1.0)


def _row_spec(w):
  return pl.BlockSpec((BLK, w), lambda i: (i, 0))


def _full_spec(shape):
  return pl.BlockSpec(shape, lambda i: tuple(0 for _ in shape))


_HB = [jax.ShapeDtypeStruct((N, HH), jnp.bfloat16)] * 2

_tc_mid = pl.pallas_call(
    _tc_mid_body,
    grid=(GRID,),
    in_specs=[_row_spec(HH)] * 4 + [_row_spec(8), _row_spec(8),
              _full_spec((8, H)), _full_spec((H, H)), _full_spec((1, H)),
              _full_spec((1, 1)), _full_spec((1, 1))],
    out_specs=[_row_spec(HH)] * 2 + [_row_spec(8)],
    out_shape=_HB + [jax.ShapeDtypeStruct((N, 8), jnp.float32)],
)

_tc_pool = pl.pallas_call(
    _tc_pool_body,
    grid=(GRID,),
    in_specs=[_row_spec(HH)] * 2 +
             [pl.BlockSpec((1, 1, BLK), lambda i: (i, 0, 0))],
    out_specs=[_row_spec(H), _full_spec((G, H))],
    out_shape=[jax.ShapeDtypeStruct((N, H), jnp.float32),
               jax.ShapeDtypeStruct((G, H), jnp.float32)],
    scratch_shapes=[pltpu.VMEM((G, H), jnp.float32),
                    pltpu.VMEM((G, 1), jnp.float32)],
)


@jax.jit
def kernel(x, edge_index, batch, W1, b1, W2, b2, W3, b3):
  src = edge_index[0]
  dst = edge_index[1]
  pad = EPAD - E
  src_i = jnp.concatenate([src, jnp.zeros((pad,), jnp.int32)])
  dst_i = jnp.concatenate([dst, jnp.full((pad,), N, jnp.int32)])
  xp = jnp.pad(x, ((0, 0), (0, 1)))
  W1p = jnp.pad(W1, ((0, 1), (0, 0)))

  ones_h = jnp.ones((N, HH), jnp.bfloat16)
  carry = (ones_h, ones_h, jnp.zeros((N, 8), jnp.float32))

  zH = jnp.zeros((H, H), jnp.float32)
  z1 = jnp.zeros((1, H), jnp.float32)
  wn_stack = jnp.stack([zH, W2, W3, zH])
  b_stack = jnp.stack([z1, b1.reshape(1, H), b2.reshape(1, H),
                       b3.reshape(1, H)])
  ff = jnp.array([1.0, 0.0, 0.0, 0.0], jnp.float32).reshape(4, 1, 1)
  lf = jnp.array([0.0, 0.0, 0.0, 1.0], jnp.float32).reshape(4, 1, 1)

  def body(carry, xs):
    wn, b, f0, f1 = xs
    sq = _sc_layer(*carry[:2], src_i, dst_i)
    nxt = _tc_mid(*sq, *carry, xp, W1p, wn, b, f0, f1)
    return tuple(nxt), None

  carry, _ = lax.scan(body, carry, (wn_stack, b_stack, ff, lf))
  h, pooled = _tc_pool(*carry[:2], batch.reshape(GRID, 1, BLK))
  return (h, pooled)
